# baseline (speedup 1.0000x reference)
import functools

import jax
import jax.numpy as jnp
from jax import lax
from jax.experimental import pallas as pl
from jax.experimental.pallas import tpu as pltpu
from jax.experimental.pallas import tpu_sc as plsc

_NC = 2
_NS = 16
_NW = _NC * _NS
_CHUNK = 16
_NBUF = 5
_LOOK = 2
_LANES = 16
_S_SC = 2560
_BLK_TC = 512


def _sc_body(batch, seq_len, s_sc, d_model, x_hbm, tab_hbm, out_hbm,
             t_bufs, x_bufs, in_sem, out_sem, t_sem):
    rows_per_w = s_sc // _NW
    c = lax.axis_index("c")
    s = lax.axis_index("s")
    wid = s * _NC + c
    base = wid * rows_per_w
    n_ci = rows_per_w // _CHUNK
    n_steps = n_ci * batch

    def rows(k):
        ci, b = divmod(k, batch)
        r = base + ci * _CHUNK
        return b * seq_len + r, b * s_sc + r

    def issue_t(ci):
        row = base + ci * _CHUNK
        return pltpu.async_copy(tab_hbm.at[pl.ds(row, _CHUNK), :],
                                t_bufs.at[ci % 2], t_sem.at[ci % 2])

    def issue_in(k):
        return pltpu.async_copy(x_hbm.at[pl.ds(rows(k)[0], _CHUNK), :],
                                x_bufs.at[k % _NBUF], in_sem.at[k % _NBUF])

    t_descs = [issue_t(0)]
    in_desc = [issue_in(k) for k in range(min(_LOOK, n_steps))]
    out_desc = []
    for k in range(n_steps):
        ci, b = divmod(k, batch)
        buf = k % _NBUF
        if b == 0 and ci + 1 < n_ci:
            t_descs.append(issue_t(ci + 1))
        if k + _LOOK < n_steps:
            reuse = k + _LOOK - _NBUF
            if reuse >= 0:
                out_desc[reuse].wait()
            in_desc.append(issue_in(k + _LOOK))
        in_desc[k].wait()
        if b == 0:
            t_descs[ci].wait()
        t_buf = t_bufs.at[ci % 2]
        x_buf = x_bufs.at[buf]
        vecs_per_row = d_model // _LANES

        @plsc.parallel_loop(0, _CHUNK * vecs_per_row, unroll=8)
        def _add(i):
            r = i // vecs_per_row
            off = (i % vecs_per_row) * _LANES
            plsc.addupdate(x_buf.at[r, pl.ds(off, _LANES)],
                           t_buf[r, pl.ds(off, _LANES)])

        out_desc.append(
            pltpu.async_copy(x_bufs.at[buf],
                             out_hbm.at[pl.ds(rows(k)[1], _CHUNK), :],
                             out_sem.at[buf]))
    for k in range(max(0, n_steps - _NBUF), n_steps):
        out_desc[k].wait()


def _tc_body(x_ref, t_ref, o_ref):
    o_ref[...] = x_ref[...] + t_ref[...][None, :, :]


def kernel(x, pos_table):
    batch, seq_len, d_model = x.shape
    s_sc = _S_SC
    x2 = x.reshape(batch * seq_len, d_model)

    mesh = plsc.VectorSubcoreMesh(core_axis_name="c", subcore_axis_name="s")
    sc_add = pl.kernel(
        functools.partial(_sc_body, batch, seq_len, s_sc, d_model),
        out_type=jax.ShapeDtypeStruct((batch * s_sc, d_model), x.dtype),
        mesh=mesh,
        scratch_types=[
            pltpu.VMEM((2, _CHUNK, d_model), jnp.float32),
            pltpu.VMEM((_NBUF, _CHUNK, d_model), jnp.float32),
            pltpu.SemaphoreType.DMA((_NBUF,)),
            pltpu.SemaphoreType.DMA((_NBUF,)),
            pltpu.SemaphoreType.DMA((2,)),
        ],
    )
    sc_out = sc_add(x2, pos_table)

    off = s_sc // _BLK_TC
    tc_out = pl.pallas_call(
        _tc_body,
        grid=((seq_len - s_sc) // _BLK_TC, ),
        in_specs=[
            pl.BlockSpec((batch, _BLK_TC, d_model), lambda i: (0, i + off, 0)),
            pl.BlockSpec((_BLK_TC, d_model), lambda i: (i + off, 0)),
        ],
        out_specs=pl.BlockSpec((batch, _BLK_TC, d_model),
                               lambda i: (0, i + off, 0)),
        out_shape=jax.ShapeDtypeStruct((batch, seq_len, d_model), x.dtype),
    )(x, pos_table)

    return lax.dynamic_update_slice(
        tc_out, sc_out.reshape(batch, s_sc, d_model), (0, 0, 0))


# speedup vs baseline: 1.0747x; 1.0747x over previous
import functools

import jax
import jax.numpy as jnp
from jax import lax
from jax.experimental import pallas as pl
from jax.experimental.pallas import tpu as pltpu
from jax.experimental.pallas import tpu_sc as plsc

_NC = 2
_NS = 16
_NW = _NC * _NS
_CHUNK = 16
_NBUF = 5
_LOOK = 2
_LANES = 16


def _sc_body(batch, seq_len, d_model, x_hbm, tab_hbm, out_hbm,
             t_bufs, x_bufs, in_sem, out_sem, t_sem):
    rows_per_w = seq_len // _NW
    c = lax.axis_index("c")
    s = lax.axis_index("s")
    wid = s * _NC + c
    base = wid * rows_per_w
    n_ci = rows_per_w // _CHUNK
    n_steps = n_ci * batch

    def xrow(k):
        ci, b = divmod(k, batch)
        return b * seq_len + base + ci * _CHUNK

    def issue_t(ci):
        row = base + ci * _CHUNK
        return pltpu.async_copy(tab_hbm.at[pl.ds(row, _CHUNK), :],
                                t_bufs.at[ci % 2], t_sem.at[ci % 2])

    def issue_in(k):
        return pltpu.async_copy(x_hbm.at[pl.ds(xrow(k), _CHUNK), :],
                                x_bufs.at[k % _NBUF], in_sem.at[k % _NBUF])

    t_descs = [issue_t(0)]
    in_desc = [issue_in(k) for k in range(min(_LOOK, n_steps))]
    out_desc = []
    for k in range(n_steps):
        ci, b = divmod(k, batch)
        buf = k % _NBUF
        if b == 0 and ci + 1 < n_ci:
            t_descs.append(issue_t(ci + 1))
        if k + _LOOK < n_steps:
            reuse = k + _LOOK - _NBUF
            if reuse >= 0:
                out_desc[reuse].wait()
            in_desc.append(issue_in(k + _LOOK))
        in_desc[k].wait()
        if b == 0:
            t_descs[ci].wait()
        t_buf = t_bufs.at[ci % 2]
        x_buf = x_bufs.at[buf]
        vecs_per_row = d_model // _LANES

        @plsc.parallel_loop(0, _CHUNK * vecs_per_row, unroll=8)
        def _add(i):
            r = i // vecs_per_row
            off = (i % vecs_per_row) * _LANES
            plsc.addupdate(x_buf.at[r, pl.ds(off, _LANES)],
                           t_buf[r, pl.ds(off, _LANES)])

        out_desc.append(
            pltpu.async_copy(x_bufs.at[buf],
                             out_hbm.at[pl.ds(xrow(k), _CHUNK), :],
                             out_sem.at[buf]))
    for k in range(max(0, n_steps - _NBUF), n_steps):
        out_desc[k].wait()


def kernel(x, pos_table):
    batch, seq_len, d_model = x.shape
    x2 = x.reshape(batch * seq_len, d_model)
    mesh = plsc.VectorSubcoreMesh(core_axis_name="c", subcore_axis_name="s")
    sc_add = pl.kernel(
        functools.partial(_sc_body, batch, seq_len, d_model),
        out_type=jax.ShapeDtypeStruct((batch * seq_len, d_model), x.dtype),
        mesh=mesh,
        scratch_types=[
            pltpu.VMEM((2, _CHUNK, d_model), jnp.float32),
            pltpu.VMEM((_NBUF, _CHUNK, d_model), jnp.float32),
            pltpu.SemaphoreType.DMA((_NBUF,)),
            pltpu.SemaphoreType.DMA((_NBUF,)),
            pltpu.SemaphoreType.DMA((2,)),
        ],
    )
    out = sc_add(x2, pos_table)
    return out.reshape(batch, seq_len, d_model)
